# baseline (speedup 1.0000x reference)
import functools

import jax
import jax.numpy as jnp
from jax import lax
from jax.experimental import pallas as pl
from jax.experimental.pallas import tpu as pltpu
from jax.experimental.pallas import tpu_sc as plsc

N = 10000
E = 320000
D = 128
DA = 144
G = 128
C = 2

NC, NS = 2, 16
NW = NC * NS
EPW = E // NW
EC = 80
NCH = EPW // EC

def _build_sc_aggregate(W, nblk, depth):
  mesh = plsc.VectorSubcoreMesh(core_axis_name="c", subcore_axis_name="s")
  ib = NCH // nblk

  scratch = ([pltpu.VMEM((EC, W), jnp.float32) for _ in range(depth)] +
             [pltpu.VMEM((ib, EC), jnp.int32),
              pltpu.VMEM((ib, EC), jnp.int32),
              pltpu.VMEM_SHARED((N, W), jnp.float32)] +
             [pltpu.SemaphoreType.DMA for _ in range(2 * depth)])

  @functools.partial(
      pl.kernel,
      mesh=mesh,
      compiler_params=pltpu.CompilerParams(use_tc_tiling_on_sc=False),
      out_type=jax.ShapeDtypeStruct((NC * N, W), jnp.float32),
      scratch_types=scratch,
  )
  def sc_agg(xa, src2, dst2, zeros, out, *scr):
    rows = scr[:depth]
    srcb, dstb, agg_sh = scr[depth:depth + 3]
    gsems = scr[depth + 3:2 * depth + 3]
    ssems = scr[2 * depth + 3:]
    cid = lax.axis_index("c")
    sid = lax.axis_index("s")
    wid = sid * NC + cid
    crow0 = wid * NCH

    r0 = sid * 624
    pltpu.sync_copy(zeros.at[pl.ds(0, 624)], agg_sh.at[pl.ds(r0, 624)])

    @pl.when(sid == 15)
    def _():
      pltpu.sync_copy(zeros.at[pl.ds(0, 16)], agg_sh.at[pl.ds(9984, 16)])

    plsc.subcore_barrier()

    def gather_start(b, j):
      pltpu.async_copy(xa.at[srcb.at[j]], rows[b], gsems[b])

    def gather_wait(b, j):
      pltpu.make_async_copy(xa.at[srcb.at[j]], rows[b], gsems[b]).wait()

    def scatter_start(b, j):
      pltpu.async_copy(rows[b], agg_sh.at[dstb.at[j]], ssems[b], add=True)

    def scatter_wait(b, j):
      pltpu.make_async_copy(rows[b], agg_sh.at[dstb.at[j]], ssems[b]).wait()

    for blk in range(nblk):
      base = crow0 + blk * ib
      pltpu.sync_copy(src2.at[pl.ds(base, ib)], srcb)
      pltpu.sync_copy(dst2.at[pl.ds(base, ib)], dstb)

      for k in range(depth - 1):
        gather_start(k, k)

      def body(j, carry):
        for par in range(depth):
          @pl.when((j % depth) == par)
          def _():
            nb = (par + depth - 1) % depth

            @pl.when(j + depth - 1 < ib)
            def _():
              @pl.when(j >= 1)
              def _():
                scatter_wait(nb, j - 1)
              gather_start(nb, j + depth - 1)

            gather_wait(par, j)
            scatter_start(par, j)
        return carry

      lax.fori_loop(0, ib, body, 0)
      for k in range(ib - depth, ib):
        scatter_wait(k % depth, k)

    plsc.subcore_barrier()

    ob = cid * N + r0
    pltpu.sync_copy(agg_sh.at[pl.ds(r0, 624)], out.at[pl.ds(ob, 624)])

    @pl.when(sid == 15)
    def _():
      pltpu.sync_copy(agg_sh.at[pl.ds(9984, 16)], out.at[pl.ds(cid * N + 9984, 16)])

  return sc_agg


_sc_aggregate_aug = _build_sc_aggregate(DA, 5, 3)
_sc_aggregate = _build_sc_aggregate(D, 5, 4)

_BR = 2000
_GRID = N // _BR
_DOT = dict(preferred_element_type=jnp.float32, precision=lax.Precision.HIGHEST)


def _layer1_body(agg_ref, x_ref, wl_ref, bl_ref, wr_ref, g_ref, be_ref,
                 h_ref, invc_ref):
  a = agg_ref[0] + agg_ref[1]
  inv = 1.0 / jnp.maximum(a[:, D:D + 1], 1.0)
  mean = a[:, :D] * inv
  xs = x_ref[...]
  h = lax.dot_general(mean, wl_ref[...], (((1,), (1,)), ((), ())), **_DOT)
  h = h + lax.dot_general(xs, wr_ref[...], (((1,), (1,)), ((), ())), **_DOT)
  h = jnp.maximum(h + bl_ref[...], 0.0)
  mu = jnp.mean(h, axis=1, keepdims=True)
  var = jnp.mean((h - mu) ** 2, axis=1, keepdims=True)
  h_ref[...] = (h - mu) * lax.rsqrt(var + 1e-5) * g_ref[...] + be_ref[...]
  invc_ref[...] = jnp.broadcast_to(inv, (_BR, D))


_tc_layer1 = pl.pallas_call(
    _layer1_body,
    grid=(_GRID,),
    in_specs=[
        pl.BlockSpec((2, _BR, DA), lambda i: (0, i, 0)),
        pl.BlockSpec((_BR, D), lambda i: (i, 0)),
        pl.BlockSpec((D, D), lambda i: (0, 0)),
        pl.BlockSpec((1, D), lambda i: (0, 0)),
        pl.BlockSpec((D, D), lambda i: (0, 0)),
        pl.BlockSpec((1, D), lambda i: (0, 0)),
        pl.BlockSpec((1, D), lambda i: (0, 0)),
    ],
    out_specs=[
        pl.BlockSpec((_BR, D), lambda i: (i, 0)),
        pl.BlockSpec((_BR, D), lambda i: (i, 0)),
    ],
    out_shape=[
        jax.ShapeDtypeStruct((N, D), jnp.float32),
        jax.ShapeDtypeStruct((N, D), jnp.float32),
    ],
)


def _layer2_body(agg_ref, x_ref, invc_ref, wl_ref, bl_ref, wr_ref, g_ref,
                 be_ref, h_ref):
  mean = (agg_ref[0] + agg_ref[1]) * invc_ref[...]
  xs = x_ref[...]
  h = lax.dot_general(mean, wl_ref[...], (((1,), (1,)), ((), ())), **_DOT)
  h = h + lax.dot_general(xs, wr_ref[...], (((1,), (1,)), ((), ())), **_DOT)
  h = jnp.maximum(h + bl_ref[...], 0.0) + xs
  mu = jnp.mean(h, axis=1, keepdims=True)
  var = jnp.mean((h - mu) ** 2, axis=1, keepdims=True)
  h_ref[...] = (h - mu) * lax.rsqrt(var + 1e-5) * g_ref[...] + be_ref[...]


_tc_layer2 = pl.pallas_call(
    _layer2_body,
    grid=(_GRID,),
    in_specs=[
        pl.BlockSpec((2, _BR, D), lambda i: (0, i, 0)),
        pl.BlockSpec((_BR, D), lambda i: (i, 0)),
        pl.BlockSpec((_BR, D), lambda i: (i, 0)),
        pl.BlockSpec((D, D), lambda i: (0, 0)),
        pl.BlockSpec((1, D), lambda i: (0, 0)),
        pl.BlockSpec((D, D), lambda i: (0, 0)),
        pl.BlockSpec((1, D), lambda i: (0, 0)),
        pl.BlockSpec((1, D), lambda i: (0, 0)),
    ],
    out_specs=pl.BlockSpec((_BR, D), lambda i: (i, 0)),
    out_shape=jax.ShapeDtypeStruct((N, D), jnp.float32),
)


def _layer3_pool_body(agg_ref, x_ref, invc_ref, wl_ref, bl_ref, wr_ref,
                      g_ref, be_ref, b_ref, wc_ref, bc_ref, h_ref, out_ref,
                      avg_ref, gsum, gcnt):
  i = pl.program_id(0)

  mean = (agg_ref[0] + agg_ref[1]) * invc_ref[...]
  xs = x_ref[...]
  h = lax.dot_general(mean, wl_ref[...], (((1,), (1,)), ((), ())), **_DOT)
  h = h + lax.dot_general(xs, wr_ref[...], (((1,), (1,)), ((), ())), **_DOT)
  h = jnp.maximum(h + bl_ref[...], 0.0) + xs
  mu = jnp.mean(h, axis=1, keepdims=True)
  var = jnp.mean((h - mu) ** 2, axis=1, keepdims=True)
  y = (h - mu) * lax.rsqrt(var + 1e-5) * g_ref[...] + be_ref[...]
  h_ref[...] = y

  @pl.when(i == 0)
  def _():
    gsum[...] = jnp.zeros_like(gsum)
    gcnt[...] = jnp.zeros_like(gcnt)

  oh = (b_ref[...] == lax.broadcasted_iota(jnp.int32, (_BR, G), 1))
  oh = oh.astype(jnp.float32)
  gsum[...] += lax.dot_general(oh, y, (((0,), (0,)), ((), ())), **_DOT)
  gcnt[...] += lax.dot_general(oh, jnp.ones((_BR, G), jnp.float32),
                               (((0,), (0,)), ((), ())), **_DOT)

  @pl.when(i == _GRID - 1)
  def _():
    avg = gsum[...] / jnp.maximum(gcnt[...], 1.0)
    avg_ref[...] = avg
    out_ref[...] = lax.dot_general(avg, wc_ref[...], (((1,), (1,)), ((), ())),
                                   **_DOT) + bc_ref[...]


_tc_layer3_pool = pl.pallas_call(
    _layer3_pool_body,
    grid=(_GRID,),
    in_specs=[
        pl.BlockSpec((2, _BR, D), lambda i: (0, i, 0)),
        pl.BlockSpec((_BR, D), lambda i: (i, 0)),
        pl.BlockSpec((_BR, D), lambda i: (i, 0)),
        pl.BlockSpec((D, D), lambda i: (0, 0)),
        pl.BlockSpec((1, D), lambda i: (0, 0)),
        pl.BlockSpec((D, D), lambda i: (0, 0)),
        pl.BlockSpec((1, D), lambda i: (0, 0)),
        pl.BlockSpec((1, D), lambda i: (0, 0)),
        pl.BlockSpec((_BR, 1), lambda i: (i, 0)),
        pl.BlockSpec((C, D), lambda i: (0, 0)),
        pl.BlockSpec((1, C), lambda i: (0, 0)),
    ],
    out_specs=[
        pl.BlockSpec((_BR, D), lambda i: (i, 0)),
        pl.BlockSpec((G, C), lambda i: (0, 0)),
        pl.BlockSpec((G, D), lambda i: (0, 0)),
    ],
    out_shape=[
        jax.ShapeDtypeStruct((N, D), jnp.float32),
        jax.ShapeDtypeStruct((G, C), jnp.float32),
        jax.ShapeDtypeStruct((G, D), jnp.float32),
    ],
    scratch_shapes=[
        pltpu.VMEM((G, D), jnp.float32),
        pltpu.VMEM((G, G), jnp.float32),
    ],
)


def kernel(x, edge_index, batch, W1l, b1l, W1r, g1, be1, W2l, b2l, W2r, g2,
           be2, W3l, b3l, W3r, g3, be3, Wc, bc):
  src2 = edge_index[0].reshape(E // EC, EC)
  dst2 = edge_index[1].reshape(E // EC, EC)
  zeros_a = jnp.zeros((624, DA), jnp.float32)
  zeros_d = jnp.zeros((624, D), jnp.float32)
  xa = jnp.concatenate(
      [x, jnp.ones((N, 1), jnp.float32), jnp.zeros((N, DA - D - 1), jnp.float32)],
      axis=1)

  agg1 = _sc_aggregate_aug(xa, src2, dst2, zeros_a).reshape(2, N, DA)
  h1, invc = _tc_layer1(agg1, x, W1l, b1l.reshape(1, D), W1r,
                        g1.reshape(1, D), be1.reshape(1, D))

  agg2 = _sc_aggregate(h1, src2, dst2, zeros_d).reshape(2, N, D)
  h2 = _tc_layer2(agg2, h1, invc, W2l, b2l.reshape(1, D), W2r,
                  g2.reshape(1, D), be2.reshape(1, D))

  agg3 = _sc_aggregate(h2, src2, dst2, zeros_d).reshape(2, N, D)
  h3, out, avg = _tc_layer3_pool(agg3, h2, invc, W3l, b3l.reshape(1, D),
                                 W3r, g3.reshape(1, D), be3.reshape(1, D),
                                 batch.reshape(N, 1), Wc, bc.reshape(1, C))
  return (out, h3, avg)


# speedup vs baseline: 1.0446x; 1.0446x over previous
import functools

import jax
import jax.numpy as jnp
from jax import lax
from jax.experimental import pallas as pl
from jax.experimental.pallas import tpu as pltpu
from jax.experimental.pallas import tpu_sc as plsc

N = 10000
E = 320000
D = 128
DA = 144
G = 128
C = 2

NC, NS = 2, 16
NW = NC * NS
EPW = E // NW
EC = 80
NCH = EPW // EC

def _build_sc_aggregate(W, nblk, depth):
  mesh = plsc.VectorSubcoreMesh(core_axis_name="c", subcore_axis_name="s")
  ib = NCH // nblk

  scratch = ([pltpu.VMEM((EC, W), jnp.float32) for _ in range(depth)] +
             [pltpu.VMEM((ib, EC), jnp.int32),
              pltpu.VMEM((ib, EC), jnp.int32),
              pltpu.VMEM_SHARED((N, W), jnp.float32)] +
             [pltpu.SemaphoreType.DMA for _ in range(2 * depth)])

  @functools.partial(
      pl.kernel,
      mesh=mesh,
      compiler_params=pltpu.CompilerParams(use_tc_tiling_on_sc=False),
      out_type=jax.ShapeDtypeStruct((NC * N, W), jnp.float32),
      scratch_types=scratch,
  )
  def sc_agg(xa, src2, dst2, zeros, out, *scr):
    rows = scr[:depth]
    srcb, dstb, agg_sh = scr[depth:depth + 3]
    gsems = scr[depth + 3:2 * depth + 3]
    ssems = scr[2 * depth + 3:]
    cid = lax.axis_index("c")
    sid = lax.axis_index("s")
    wid = sid * NC + cid
    crow0 = wid * NCH

    r0 = sid * 624
    pltpu.sync_copy(zeros.at[pl.ds(0, 624)], agg_sh.at[pl.ds(r0, 624)])

    @pl.when(sid == 15)
    def _():
      pltpu.sync_copy(zeros.at[pl.ds(0, 16)], agg_sh.at[pl.ds(9984, 16)])

    plsc.subcore_barrier()

    def gather_start(b, j):
      pltpu.async_copy(xa.at[srcb.at[j]], rows[b], gsems[b])

    def gather_wait(b, j):
      pltpu.make_async_copy(xa.at[srcb.at[j]], rows[b], gsems[b]).wait()

    def scatter_start(b, j):
      pltpu.async_copy(rows[b], agg_sh.at[dstb.at[j]], ssems[b], add=True)

    def scatter_wait(b, j):
      pltpu.make_async_copy(rows[b], agg_sh.at[dstb.at[j]], ssems[b]).wait()

    for blk in range(nblk):
      base = crow0 + blk * ib
      pltpu.sync_copy(src2.at[pl.ds(base, ib)], srcb)
      pltpu.sync_copy(dst2.at[pl.ds(base, ib)], dstb)

      for k in range(depth - 1):
        gather_start(k, k)

      def body(j, carry):
        for par in range(depth):
          @pl.when((j % depth) == par)
          def _():
            nb = (par + depth - 1) % depth

            @pl.when(j + depth - 1 < ib)
            def _():
              @pl.when(j >= 1)
              def _():
                scatter_wait(nb, j - 1)
              gather_start(nb, j + depth - 1)

            gather_wait(par, j)
            scatter_start(par, j)
        return carry

      lax.fori_loop(0, ib, body, 0)
      for k in range(ib - depth, ib):
        scatter_wait(k % depth, k)

    plsc.subcore_barrier()

    ob = cid * N + r0
    pltpu.sync_copy(agg_sh.at[pl.ds(r0, 624)], out.at[pl.ds(ob, 624)])

    @pl.when(sid == 15)
    def _():
      pltpu.sync_copy(agg_sh.at[pl.ds(9984, 16)], out.at[pl.ds(cid * N + 9984, 16)])

  return sc_agg


_sc_aggregate_aug = _build_sc_aggregate(DA, 5, 3)
_sc_aggregate = _build_sc_aggregate(D, 1, 3)

_BR = 2000
_GRID = N // _BR
_DOT = dict(preferred_element_type=jnp.float32, precision=lax.Precision.HIGHEST)


def _layer1_body(agg_ref, x_ref, wl_ref, bl_ref, wr_ref, g_ref, be_ref,
                 h_ref, invc_ref):
  a = agg_ref[0] + agg_ref[1]
  inv = 1.0 / jnp.maximum(a[:, D:D + 1], 1.0)
  mean = a[:, :D] * inv
  xs = x_ref[...]
  h = lax.dot_general(mean, wl_ref[...], (((1,), (1,)), ((), ())), **_DOT)
  h = h + lax.dot_general(xs, wr_ref[...], (((1,), (1,)), ((), ())), **_DOT)
  h = jnp.maximum(h + bl_ref[...], 0.0)
  mu = jnp.mean(h, axis=1, keepdims=True)
  var = jnp.mean((h - mu) ** 2, axis=1, keepdims=True)
  h_ref[...] = (h - mu) * lax.rsqrt(var + 1e-5) * g_ref[...] + be_ref[...]
  invc_ref[...] = jnp.broadcast_to(inv, (_BR, D))


_tc_layer1 = pl.pallas_call(
    _layer1_body,
    grid=(_GRID,),
    in_specs=[
        pl.BlockSpec((2, _BR, DA), lambda i: (0, i, 0)),
        pl.BlockSpec((_BR, D), lambda i: (i, 0)),
        pl.BlockSpec((D, D), lambda i: (0, 0)),
        pl.BlockSpec((1, D), lambda i: (0, 0)),
        pl.BlockSpec((D, D), lambda i: (0, 0)),
        pl.BlockSpec((1, D), lambda i: (0, 0)),
        pl.BlockSpec((1, D), lambda i: (0, 0)),
    ],
    out_specs=[
        pl.BlockSpec((_BR, D), lambda i: (i, 0)),
        pl.BlockSpec((_BR, D), lambda i: (i, 0)),
    ],
    out_shape=[
        jax.ShapeDtypeStruct((N, D), jnp.float32),
        jax.ShapeDtypeStruct((N, D), jnp.float32),
    ],
)


def _layer2_body(agg_ref, x_ref, invc_ref, wl_ref, bl_ref, wr_ref, g_ref,
                 be_ref, h_ref):
  mean = (agg_ref[0] + agg_ref[1]) * invc_ref[...]
  xs = x_ref[...]
  h = lax.dot_general(mean, wl_ref[...], (((1,), (1,)), ((), ())), **_DOT)
  h = h + lax.dot_general(xs, wr_ref[...], (((1,), (1,)), ((), ())), **_DOT)
  h = jnp.maximum(h + bl_ref[...], 0.0) + xs
  mu = jnp.mean(h, axis=1, keepdims=True)
  var = jnp.mean((h - mu) ** 2, axis=1, keepdims=True)
  h_ref[...] = (h - mu) * lax.rsqrt(var + 1e-5) * g_ref[...] + be_ref[...]


_tc_layer2 = pl.pallas_call(
    _layer2_body,
    grid=(_GRID,),
    in_specs=[
        pl.BlockSpec((2, _BR, D), lambda i: (0, i, 0)),
        pl.BlockSpec((_BR, D), lambda i: (i, 0)),
        pl.BlockSpec((_BR, D), lambda i: (i, 0)),
        pl.BlockSpec((D, D), lambda i: (0, 0)),
        pl.BlockSpec((1, D), lambda i: (0, 0)),
        pl.BlockSpec((D, D), lambda i: (0, 0)),
        pl.BlockSpec((1, D), lambda i: (0, 0)),
        pl.BlockSpec((1, D), lambda i: (0, 0)),
    ],
    out_specs=pl.BlockSpec((_BR, D), lambda i: (i, 0)),
    out_shape=jax.ShapeDtypeStruct((N, D), jnp.float32),
)


def _layer3_pool_body(agg_ref, x_ref, invc_ref, wl_ref, bl_ref, wr_ref,
                      g_ref, be_ref, b_ref, wc_ref, bc_ref, h_ref, out_ref,
                      avg_ref, gsum, gcnt):
  i = pl.program_id(0)

  mean = (agg_ref[0] + agg_ref[1]) * invc_ref[...]
  xs = x_ref[...]
  h = lax.dot_general(mean, wl_ref[...], (((1,), (1,)), ((), ())), **_DOT)
  h = h + lax.dot_general(xs, wr_ref[...], (((1,), (1,)), ((), ())), **_DOT)
  h = jnp.maximum(h + bl_ref[...], 0.0) + xs
  mu = jnp.mean(h, axis=1, keepdims=True)
  var = jnp.mean((h - mu) ** 2, axis=1, keepdims=True)
  y = (h - mu) * lax.rsqrt(var + 1e-5) * g_ref[...] + be_ref[...]
  h_ref[...] = y

  @pl.when(i == 0)
  def _():
    gsum[...] = jnp.zeros_like(gsum)
    gcnt[...] = jnp.zeros_like(gcnt)

  oh = (b_ref[...] == lax.broadcasted_iota(jnp.int32, (_BR, G), 1))
  oh = oh.astype(jnp.float32)
  gsum[...] += lax.dot_general(oh, y, (((0,), (0,)), ((), ())), **_DOT)
  gcnt[...] += lax.dot_general(oh, jnp.ones((_BR, G), jnp.float32),
                               (((0,), (0,)), ((), ())), **_DOT)

  @pl.when(i == _GRID - 1)
  def _():
    avg = gsum[...] / jnp.maximum(gcnt[...], 1.0)
    avg_ref[...] = avg
    out_ref[...] = lax.dot_general(avg, wc_ref[...], (((1,), (1,)), ((), ())),
                                   **_DOT) + bc_ref[...]


_tc_layer3_pool = pl.pallas_call(
    _layer3_pool_body,
    grid=(_GRID,),
    in_specs=[
        pl.BlockSpec((2, _BR, D), lambda i: (0, i, 0)),
        pl.BlockSpec((_BR, D), lambda i: (i, 0)),
        pl.BlockSpec((_BR, D), lambda i: (i, 0)),
        pl.BlockSpec((D, D), lambda i: (0, 0)),
        pl.BlockSpec((1, D), lambda i: (0, 0)),
        pl.BlockSpec((D, D), lambda i: (0, 0)),
        pl.BlockSpec((1, D), lambda i: (0, 0)),
        pl.BlockSpec((1, D), lambda i: (0, 0)),
        pl.BlockSpec((_BR, 1), lambda i: (i, 0)),
        pl.BlockSpec((C, D), lambda i: (0, 0)),
        pl.BlockSpec((1, C), lambda i: (0, 0)),
    ],
    out_specs=[
        pl.BlockSpec((_BR, D), lambda i: (i, 0)),
        pl.BlockSpec((G, C), lambda i: (0, 0)),
        pl.BlockSpec((G, D), lambda i: (0, 0)),
    ],
    out_shape=[
        jax.ShapeDtypeStruct((N, D), jnp.float32),
        jax.ShapeDtypeStruct((G, C), jnp.float32),
        jax.ShapeDtypeStruct((G, D), jnp.float32),
    ],
    scratch_shapes=[
        pltpu.VMEM((G, D), jnp.float32),
        pltpu.VMEM((G, G), jnp.float32),
    ],
)


def kernel(x, edge_index, batch, W1l, b1l, W1r, g1, be1, W2l, b2l, W2r, g2,
           be2, W3l, b3l, W3r, g3, be3, Wc, bc):
  src2 = edge_index[0].reshape(E // EC, EC)
  dst2 = edge_index[1].reshape(E // EC, EC)
  zeros_a = jnp.zeros((624, DA), jnp.float32)
  zeros_d = jnp.zeros((624, D), jnp.float32)
  xa = jnp.concatenate(
      [x, jnp.ones((N, 1), jnp.float32), jnp.zeros((N, DA - D - 1), jnp.float32)],
      axis=1)

  agg1 = _sc_aggregate_aug(xa, src2, dst2, zeros_a).reshape(2, N, DA)
  h1, invc = _tc_layer1(agg1, x, W1l, b1l.reshape(1, D), W1r,
                        g1.reshape(1, D), be1.reshape(1, D))

  agg2 = _sc_aggregate(h1, src2, dst2, zeros_d).reshape(2, N, D)
  h2 = _tc_layer2(agg2, h1, invc, W2l, b2l.reshape(1, D), W2r,
                  g2.reshape(1, D), be2.reshape(1, D))

  agg3 = _sc_aggregate(h2, src2, dst2, zeros_d).reshape(2, N, D)
  h3, out, avg = _tc_layer3_pool(agg3, h2, invc, W3l, b3l.reshape(1, D),
                                 W3r, g3.reshape(1, D), be3.reshape(1, D),
                                 batch.reshape(N, 1), Wc, bc.reshape(1, C))
  return (out, h3, avg)


# speedup vs baseline: 1.1160x; 1.0683x over previous
import functools

import jax
import jax.numpy as jnp
from jax import lax
from jax.experimental import pallas as pl
from jax.experimental.pallas import tpu as pltpu
from jax.experimental.pallas import tpu_sc as plsc

N = 10000
E = 320000
D = 128
DA = 144
G = 128
C = 2

NC, NS = 2, 16
NW = NC * NS
EPW = E // NW
EC = 80
NCH = EPW // EC

_CB = 48


def _build_sc_aggregate(W, nblk, depth, counts=False):
  mesh = plsc.VectorSubcoreMesh(core_axis_name="c", subcore_axis_name="s")
  ib = NCH // nblk

  scratch = ([pltpu.VMEM((EC, W), jnp.float32) for _ in range(depth)] +
             [pltpu.VMEM((ib, EC), jnp.int32),
              pltpu.VMEM((ib, EC), jnp.int32),
              pltpu.VMEM_SHARED((N, W), jnp.float32)] +
             [pltpu.SemaphoreType.DMA for _ in range(2 * depth)])
  out_type = [jax.ShapeDtypeStruct((NC * N, W), jnp.float32)]
  if counts:
    scratch += [
        pltpu.VMEM_SHARED((N, 16), jnp.float32),
        pltpu.VMEM((EC, 16), jnp.float32),
        pltpu.VMEM((_CB, 16), jnp.float32),
        pltpu.SemaphoreType.DMA,
    ]
    out_type += [jax.ShapeDtypeStruct((NC * N, D), jnp.float32)]

  @functools.partial(
      pl.kernel,
      mesh=mesh,
      compiler_params=pltpu.CompilerParams(use_tc_tiling_on_sc=False),
      out_type=out_type,
      scratch_types=scratch,
  )
  def sc_agg(xa, src2, dst2, zeros, *rest):
    if counts:
      zeros16, ones_h, out, cntout = rest[0], rest[1], rest[2], rest[3]
      scr = rest[4:]
      cnt_sh, ones_v, cb, csem = scr[3 * depth + 3:]
    else:
      out = rest[0]
      scr = rest[1:]
    rows = scr[:depth]
    srcb, dstb, agg_sh = scr[depth:depth + 3]
    gsems = scr[depth + 3:2 * depth + 3]
    ssems = scr[2 * depth + 3:3 * depth + 3]
    cid = lax.axis_index("c")
    sid = lax.axis_index("s")
    wid = sid * NC + cid
    crow0 = wid * NCH

    r0 = sid * 624
    pltpu.sync_copy(zeros.at[pl.ds(0, 624)], agg_sh.at[pl.ds(r0, 624)])

    @pl.when(sid == 15)
    def _():
      pltpu.sync_copy(zeros.at[pl.ds(0, 16)], agg_sh.at[pl.ds(9984, 16)])

    if counts:
      pltpu.sync_copy(zeros16.at[pl.ds(0, 624)], cnt_sh.at[pl.ds(r0, 624)])
      pltpu.sync_copy(ones_h, ones_v)

      @pl.when(sid == 15)
      def _():
        pltpu.sync_copy(zeros16.at[pl.ds(0, 16)], cnt_sh.at[pl.ds(9984, 16)])

    plsc.subcore_barrier()

    def gather_start(b, j):
      pltpu.async_copy(xa.at[srcb.at[j]], rows[b], gsems[b])

    def gather_wait(b, j):
      pltpu.make_async_copy(xa.at[srcb.at[j]], rows[b], gsems[b]).wait()

    def scatter_start(b, j):
      pltpu.async_copy(rows[b], agg_sh.at[dstb.at[j]], ssems[b], add=True)

    def scatter_wait(b, j):
      pltpu.make_async_copy(rows[b], agg_sh.at[dstb.at[j]], ssems[b]).wait()

    for blk in range(nblk):
      base = crow0 + blk * ib
      pltpu.sync_copy(src2.at[pl.ds(base, ib)], srcb)
      pltpu.sync_copy(dst2.at[pl.ds(base, ib)], dstb)

      for k in range(depth - 1):
        gather_start(k, k)

      def body(j, carry):
        for par in range(depth):
          @pl.when((j % depth) == par)
          def _():
            nb = (par + depth - 1) % depth

            @pl.when(j + depth - 1 < ib)
            def _():
              @pl.when(j >= 1)
              def _():
                scatter_wait(nb, j - 1)
              gather_start(nb, j + depth - 1)

            gather_wait(par, j)
            scatter_start(par, j)
            if counts:
              pltpu.async_copy(ones_v, cnt_sh.at[dstb.at[j]], csem, add=True)
        return carry

      lax.fori_loop(0, ib, body, 0)
      for k in range(ib - depth, ib):
        scatter_wait(k % depth, k)

    if counts:
      def cdrain(t, carry):
        pltpu.make_async_copy(ones_v, cnt_sh.at[dstb.at[0]], csem).wait()
        return carry

      lax.fori_loop(0, NCH, cdrain, 0)

    plsc.subcore_barrier()

    ob = cid * N + r0
    pltpu.sync_copy(agg_sh.at[pl.ds(r0, 624)], out.at[pl.ds(ob, 624)])

    @pl.when(sid == 15)
    def _():
      pltpu.sync_copy(agg_sh.at[pl.ds(9984, 16)], out.at[pl.ds(cid * N + 9984, 16)])

    if counts:
      def expand(nrows, cr, orow):
        pltpu.sync_copy(cnt_sh.at[pl.ds(cr, nrows)], cb.at[pl.ds(0, nrows)])

        def erow(r, c2):
          v = cb[r, :]
          for kk in range(8):
            rows[0][r, pl.ds(kk * 16, 16)] = v
          return c2

        lax.fori_loop(0, nrows, erow, 0)
        pltpu.sync_copy(rows[0].at[pl.ds(0, nrows)], cntout.at[pl.ds(orow, nrows)])

      def cexp(k, carry):
        expand(_CB, r0 + k * _CB, cid * N + r0 + k * _CB)
        return carry

      lax.fori_loop(0, 624 // _CB, cexp, 0)

      @pl.when(sid == 15)
      def _():
        expand(16, 9984, cid * N + 9984)

  return sc_agg


_sc_aggregate_cnt = _build_sc_aggregate(D, 5, 3, counts=True)
_sc_aggregate = _build_sc_aggregate(D, 1, 3)

_BR = 2000
_GRID = N // _BR
_DOT = dict(preferred_element_type=jnp.float32, precision=lax.Precision.HIGHEST)


def _layer1_body(agg_ref, cnt_ref, x_ref, wl_ref, bl_ref, wr_ref, g_ref,
                 be_ref, h_ref, invc_ref):
  inv = 1.0 / jnp.maximum(cnt_ref[0] + cnt_ref[1], 1.0)
  mean = (agg_ref[0] + agg_ref[1]) * inv
  xs = x_ref[...]
  h = lax.dot_general(mean, wl_ref[...], (((1,), (1,)), ((), ())), **_DOT)
  h = h + lax.dot_general(xs, wr_ref[...], (((1,), (1,)), ((), ())), **_DOT)
  h = jnp.maximum(h + bl_ref[...], 0.0)
  mu = jnp.mean(h, axis=1, keepdims=True)
  var = jnp.mean((h - mu) ** 2, axis=1, keepdims=True)
  h_ref[...] = (h - mu) * lax.rsqrt(var + 1e-5) * g_ref[...] + be_ref[...]
  invc_ref[...] = inv


_tc_layer1 = pl.pallas_call(
    _layer1_body,
    grid=(_GRID,),
    in_specs=[
        pl.BlockSpec((2, _BR, D), lambda i: (0, i, 0)),
        pl.BlockSpec((2, _BR, D), lambda i: (0, i, 0)),
        pl.BlockSpec((_BR, D), lambda i: (i, 0)),
        pl.BlockSpec((D, D), lambda i: (0, 0)),
        pl.BlockSpec((1, D), lambda i: (0, 0)),
        pl.BlockSpec((D, D), lambda i: (0, 0)),
        pl.BlockSpec((1, D), lambda i: (0, 0)),
        pl.BlockSpec((1, D), lambda i: (0, 0)),
    ],
    out_specs=[
        pl.BlockSpec((_BR, D), lambda i: (i, 0)),
        pl.BlockSpec((_BR, D), lambda i: (i, 0)),
    ],
    out_shape=[
        jax.ShapeDtypeStruct((N, D), jnp.float32),
        jax.ShapeDtypeStruct((N, D), jnp.float32),
    ],
)


def _layer2_body(agg_ref, x_ref, invc_ref, wl_ref, bl_ref, wr_ref, g_ref,
                 be_ref, h_ref):
  mean = (agg_ref[0] + agg_ref[1]) * invc_ref[...]
  xs = x_ref[...]
  h = lax.dot_general(mean, wl_ref[...], (((1,), (1,)), ((), ())), **_DOT)
  h = h + lax.dot_general(xs, wr_ref[...], (((1,), (1,)), ((), ())), **_DOT)
  h = jnp.maximum(h + bl_ref[...], 0.0) + xs
  mu = jnp.mean(h, axis=1, keepdims=True)
  var = jnp.mean((h - mu) ** 2, axis=1, keepdims=True)
  h_ref[...] = (h - mu) * lax.rsqrt(var + 1e-5) * g_ref[...] + be_ref[...]


_tc_layer2 = pl.pallas_call(
    _layer2_body,
    grid=(_GRID,),
    in_specs=[
        pl.BlockSpec((2, _BR, D), lambda i: (0, i, 0)),
        pl.BlockSpec((_BR, D), lambda i: (i, 0)),
        pl.BlockSpec((_BR, D), lambda i: (i, 0)),
        pl.BlockSpec((D, D), lambda i: (0, 0)),
        pl.BlockSpec((1, D), lambda i: (0, 0)),
        pl.BlockSpec((D, D), lambda i: (0, 0)),
        pl.BlockSpec((1, D), lambda i: (0, 0)),
        pl.BlockSpec((1, D), lambda i: (0, 0)),
    ],
    out_specs=pl.BlockSpec((_BR, D), lambda i: (i, 0)),
    out_shape=jax.ShapeDtypeStruct((N, D), jnp.float32),
)


def _layer3_pool_body(agg_ref, x_ref, invc_ref, wl_ref, bl_ref, wr_ref,
                      g_ref, be_ref, b_ref, wc_ref, bc_ref, h_ref, out_ref,
                      avg_ref, gsum, gcnt):
  i = pl.program_id(0)

  mean = (agg_ref[0] + agg_ref[1]) * invc_ref[...]
  xs = x_ref[...]
  h = lax.dot_general(mean, wl_ref[...], (((1,), (1,)), ((), ())), **_DOT)
  h = h + lax.dot_general(xs, wr_ref[...], (((1,), (1,)), ((), ())), **_DOT)
  h = jnp.maximum(h + bl_ref[...], 0.0) + xs
  mu = jnp.mean(h, axis=1, keepdims=True)
  var = jnp.mean((h - mu) ** 2, axis=1, keepdims=True)
  y = (h - mu) * lax.rsqrt(var + 1e-5) * g_ref[...] + be_ref[...]
  h_ref[...] = y

  @pl.when(i == 0)
  def _():
    gsum[...] = jnp.zeros_like(gsum)
    gcnt[...] = jnp.zeros_like(gcnt)

  oh = (b_ref[...] == lax.broadcasted_iota(jnp.int32, (_BR, G), 1))
  oh = oh.astype(jnp.float32)
  gsum[...] += lax.dot_general(oh, y, (((0,), (0,)), ((), ())), **_DOT)
  gcnt[...] += lax.dot_general(oh, jnp.ones((_BR, G), jnp.float32),
                               (((0,), (0,)), ((), ())), **_DOT)

  @pl.when(i == _GRID - 1)
  def _():
    avg = gsum[...] / jnp.maximum(gcnt[...], 1.0)
    avg_ref[...] = avg
    out_ref[...] = lax.dot_general(avg, wc_ref[...], (((1,), (1,)), ((), ())),
                                   **_DOT) + bc_ref[...]


_tc_layer3_pool = pl.pallas_call(
    _layer3_pool_body,
    grid=(_GRID,),
    in_specs=[
        pl.BlockSpec((2, _BR, D), lambda i: (0, i, 0)),
        pl.BlockSpec((_BR, D), lambda i: (i, 0)),
        pl.BlockSpec((_BR, D), lambda i: (i, 0)),
        pl.BlockSpec((D, D), lambda i: (0, 0)),
        pl.BlockSpec((1, D), lambda i: (0, 0)),
        pl.BlockSpec((D, D), lambda i: (0, 0)),
        pl.BlockSpec((1, D), lambda i: (0, 0)),
        pl.BlockSpec((1, D), lambda i: (0, 0)),
        pl.BlockSpec((_BR, 1), lambda i: (i, 0)),
        pl.BlockSpec((C, D), lambda i: (0, 0)),
        pl.BlockSpec((1, C), lambda i: (0, 0)),
    ],
    out_specs=[
        pl.BlockSpec((_BR, D), lambda i: (i, 0)),
        pl.BlockSpec((G, C), lambda i: (0, 0)),
        pl.BlockSpec((G, D), lambda i: (0, 0)),
    ],
    out_shape=[
        jax.ShapeDtypeStruct((N, D), jnp.float32),
        jax.ShapeDtypeStruct((G, C), jnp.float32),
        jax.ShapeDtypeStruct((G, D), jnp.float32),
    ],
    scratch_shapes=[
        pltpu.VMEM((G, D), jnp.float32),
        pltpu.VMEM((G, G), jnp.float32),
    ],
)


def kernel(x, edge_index, batch, W1l, b1l, W1r, g1, be1, W2l, b2l, W2r, g2,
           be2, W3l, b3l, W3r, g3, be3, Wc, bc):
  src2 = edge_index[0].reshape(E // EC, EC)
  dst2 = edge_index[1].reshape(E // EC, EC)
  zeros_d = jnp.zeros((624, D), jnp.float32)
  zeros16 = jnp.zeros((624, 16), jnp.float32)
  ones_h = jnp.ones((EC, 16), jnp.float32)

  agg1, cnt1 = _sc_aggregate_cnt(x, src2, dst2, zeros_d, zeros16, ones_h)
  h1, invc = _tc_layer1(agg1.reshape(2, N, D), cnt1.reshape(2, N, D), x,
                        W1l, b1l.reshape(1, D), W1r,
                        g1.reshape(1, D), be1.reshape(1, D))

  agg2 = _sc_aggregate(h1, src2, dst2, zeros_d)[0].reshape(2, N, D)
  h2 = _tc_layer2(agg2, h1, invc, W2l, b2l.reshape(1, D), W2r,
                  g2.reshape(1, D), be2.reshape(1, D))

  agg3 = _sc_aggregate(h2, src2, dst2, zeros_d)[0].reshape(2, N, D)
  h3, out, avg = _tc_layer3_pool(agg3, h2, invc, W3l, b3l.reshape(1, D),
                                 W3r, g3.reshape(1, D), be3.reshape(1, D),
                                 batch.reshape(N, 1), Wc, bc.reshape(1, C))
  return (out, h3, avg)


# speedup vs baseline: 1.1162x; 1.0002x over previous
import functools

import jax
import jax.numpy as jnp
from jax import lax
from jax.experimental import pallas as pl
from jax.experimental.pallas import tpu as pltpu
from jax.experimental.pallas import tpu_sc as plsc

N = 10000
E = 320000
D = 128
DA = 144
G = 128
C = 2

NC, NS = 2, 16
NW = NC * NS
EPW = E // NW
EC = 80
NCH = EPW // EC

_CB = 48


def _build_sc_aggregate(W, nblk, depth, counts=False):
  mesh = plsc.VectorSubcoreMesh(core_axis_name="c", subcore_axis_name="s")
  ib = NCH // nblk

  scratch = ([pltpu.VMEM((EC, W), jnp.float32) for _ in range(depth)] +
             [pltpu.VMEM((ib, EC), jnp.int32),
              pltpu.VMEM((ib, EC), jnp.int32),
              pltpu.VMEM_SHARED((N, W), jnp.float32)] +
             [pltpu.SemaphoreType.DMA for _ in range(2 * depth)])
  out_type = [jax.ShapeDtypeStruct((NC * N, W), jnp.float32)]
  if counts:
    scratch += [
        pltpu.VMEM_SHARED((N, 16), jnp.float32),
        pltpu.VMEM((EC, 16), jnp.float32),
        pltpu.VMEM((_CB, 16), jnp.float32),
        pltpu.SemaphoreType.DMA,
    ]
    out_type += [jax.ShapeDtypeStruct((NC * N, D), jnp.float32)]

  @functools.partial(
      pl.kernel,
      mesh=mesh,
      compiler_params=pltpu.CompilerParams(use_tc_tiling_on_sc=False),
      out_type=out_type,
      scratch_types=scratch,
  )
  def sc_agg(xa, src2, dst2, zeros, *rest):
    if counts:
      zeros16, ones_h, out, cntout = rest[0], rest[1], rest[2], rest[3]
      scr = rest[4:]
      cnt_sh, ones_v, cb, csem = scr[3 * depth + 3:]
    else:
      out = rest[0]
      scr = rest[1:]
    rows = scr[:depth]
    srcb, dstb, agg_sh = scr[depth:depth + 3]
    gsems = scr[depth + 3:2 * depth + 3]
    ssems = scr[2 * depth + 3:3 * depth + 3]
    cid = lax.axis_index("c")
    sid = lax.axis_index("s")
    wid = sid * NC + cid
    crow0 = wid * NCH

    r0 = sid * 624
    pltpu.sync_copy(zeros.at[pl.ds(0, 624)], agg_sh.at[pl.ds(r0, 624)])

    @pl.when(sid == 15)
    def _():
      pltpu.sync_copy(zeros.at[pl.ds(0, 16)], agg_sh.at[pl.ds(9984, 16)])

    if counts:
      pltpu.sync_copy(zeros16.at[pl.ds(0, 624)], cnt_sh.at[pl.ds(r0, 624)])
      pltpu.sync_copy(ones_h, ones_v)

      @pl.when(sid == 15)
      def _():
        pltpu.sync_copy(zeros16.at[pl.ds(0, 16)], cnt_sh.at[pl.ds(9984, 16)])

    plsc.subcore_barrier()

    def gather_start(b, j):
      pltpu.async_copy(xa.at[srcb.at[j]], rows[b], gsems[b])

    def gather_wait(b, j):
      pltpu.make_async_copy(xa.at[srcb.at[j]], rows[b], gsems[b]).wait()

    def scatter_start(b, j):
      pltpu.async_copy(rows[b], agg_sh.at[dstb.at[j]], ssems[b], add=True)

    def scatter_wait(b, j):
      pltpu.make_async_copy(rows[b], agg_sh.at[dstb.at[j]], ssems[b]).wait()

    for blk in range(nblk):
      base = crow0 + blk * ib
      pltpu.sync_copy(src2.at[pl.ds(base, ib)], srcb)
      pltpu.sync_copy(dst2.at[pl.ds(base, ib)], dstb)

      for k in range(depth - 1):
        gather_start(k, k)

      def body(j, carry):
        for par in range(depth):
          @pl.when((j % depth) == par)
          def _():
            nb = (par + depth - 1) % depth

            @pl.when(j + depth - 1 < ib)
            def _():
              @pl.when(j >= 1)
              def _():
                scatter_wait(nb, j - 1)
              gather_start(nb, j + depth - 1)

            gather_wait(par, j)
            scatter_start(par, j)
            if counts:
              pltpu.async_copy(ones_v, cnt_sh.at[dstb.at[j]], csem, add=True)
        return carry

      lax.fori_loop(0, ib, body, 0)
      for k in range(ib - depth, ib):
        scatter_wait(k % depth, k)

      if counts:
        def cdrain(t, carry):
          pltpu.make_async_copy(ones_v, cnt_sh.at[dstb.at[0]], csem).wait()
          return carry

        lax.fori_loop(0, ib, cdrain, 0)

    plsc.subcore_barrier()

    ob = cid * N + r0
    pltpu.sync_copy(agg_sh.at[pl.ds(r0, 624)], out.at[pl.ds(ob, 624)])

    @pl.when(sid == 15)
    def _():
      pltpu.sync_copy(agg_sh.at[pl.ds(9984, 16)], out.at[pl.ds(cid * N + 9984, 16)])

    if counts:
      def expand(nrows, cr, orow):
        pltpu.sync_copy(cnt_sh.at[pl.ds(cr, nrows)], cb.at[pl.ds(0, nrows)])

        def erow(r, c2):
          v = cb[r, :]
          for kk in range(8):
            rows[0][r, pl.ds(kk * 16, 16)] = v
          return c2

        lax.fori_loop(0, nrows, erow, 0)
        pltpu.sync_copy(rows[0].at[pl.ds(0, nrows)], cntout.at[pl.ds(orow, nrows)])

      def cexp(k, carry):
        expand(_CB, r0 + k * _CB, cid * N + r0 + k * _CB)
        return carry

      lax.fori_loop(0, 624 // _CB, cexp, 0)

      @pl.when(sid == 15)
      def _():
        expand(16, 9984, cid * N + 9984)

  return sc_agg


_sc_aggregate_cnt = _build_sc_aggregate(D, 5, 3, counts=True)
_sc_aggregate = _build_sc_aggregate(D, 1, 3)

_BR = 2000
_GRID = N // _BR
_DOT = dict(preferred_element_type=jnp.float32, precision=lax.Precision.HIGHEST)


def _layer1_body(agg_ref, cnt_ref, x_ref, wl_ref, bl_ref, wr_ref, g_ref,
                 be_ref, h_ref, invc_ref):
  inv = 1.0 / jnp.maximum(cnt_ref[0] + cnt_ref[1], 1.0)
  mean = (agg_ref[0] + agg_ref[1]) * inv
  xs = x_ref[...]
  h = lax.dot_general(mean, wl_ref[...], (((1,), (1,)), ((), ())), **_DOT)
  h = h + lax.dot_general(xs, wr_ref[...], (((1,), (1,)), ((), ())), **_DOT)
  h = jnp.maximum(h + bl_ref[...], 0.0)
  mu = jnp.mean(h, axis=1, keepdims=True)
  var = jnp.mean((h - mu) ** 2, axis=1, keepdims=True)
  h_ref[...] = (h - mu) * lax.rsqrt(var + 1e-5) * g_ref[...] + be_ref[...]
  invc_ref[...] = inv


_tc_layer1 = pl.pallas_call(
    _layer1_body,
    grid=(_GRID,),
    in_specs=[
        pl.BlockSpec((2, _BR, D), lambda i: (0, i, 0)),
        pl.BlockSpec((2, _BR, D), lambda i: (0, i, 0)),
        pl.BlockSpec((_BR, D), lambda i: (i, 0)),
        pl.BlockSpec((D, D), lambda i: (0, 0)),
        pl.BlockSpec((1, D), lambda i: (0, 0)),
        pl.BlockSpec((D, D), lambda i: (0, 0)),
        pl.BlockSpec((1, D), lambda i: (0, 0)),
        pl.BlockSpec((1, D), lambda i: (0, 0)),
    ],
    out_specs=[
        pl.BlockSpec((_BR, D), lambda i: (i, 0)),
        pl.BlockSpec((_BR, D), lambda i: (i, 0)),
    ],
    out_shape=[
        jax.ShapeDtypeStruct((N, D), jnp.float32),
        jax.ShapeDtypeStruct((N, D), jnp.float32),
    ],
)


def _layer2_body(agg_ref, x_ref, invc_ref, wl_ref, bl_ref, wr_ref, g_ref,
                 be_ref, h_ref):
  mean = (agg_ref[0] + agg_ref[1]) * invc_ref[...]
  xs = x_ref[...]
  h = lax.dot_general(mean, wl_ref[...], (((1,), (1,)), ((), ())), **_DOT)
  h = h + lax.dot_general(xs, wr_ref[...], (((1,), (1,)), ((), ())), **_DOT)
  h = jnp.maximum(h + bl_ref[...], 0.0) + xs
  mu = jnp.mean(h, axis=1, keepdims=True)
  var = jnp.mean((h - mu) ** 2, axis=1, keepdims=True)
  h_ref[...] = (h - mu) * lax.rsqrt(var + 1e-5) * g_ref[...] + be_ref[...]


_tc_layer2 = pl.pallas_call(
    _layer2_body,
    grid=(_GRID,),
    in_specs=[
        pl.BlockSpec((2, _BR, D), lambda i: (0, i, 0)),
        pl.BlockSpec((_BR, D), lambda i: (i, 0)),
        pl.BlockSpec((_BR, D), lambda i: (i, 0)),
        pl.BlockSpec((D, D), lambda i: (0, 0)),
        pl.BlockSpec((1, D), lambda i: (0, 0)),
        pl.BlockSpec((D, D), lambda i: (0, 0)),
        pl.BlockSpec((1, D), lambda i: (0, 0)),
        pl.BlockSpec((1, D), lambda i: (0, 0)),
    ],
    out_specs=pl.BlockSpec((_BR, D), lambda i: (i, 0)),
    out_shape=jax.ShapeDtypeStruct((N, D), jnp.float32),
)


def _layer3_pool_body(agg_ref, x_ref, invc_ref, wl_ref, bl_ref, wr_ref,
                      g_ref, be_ref, b_ref, wc_ref, bc_ref, h_ref, out_ref,
                      avg_ref, gsum, gcnt):
  i = pl.program_id(0)

  mean = (agg_ref[0] + agg_ref[1]) * invc_ref[...]
  xs = x_ref[...]
  h = lax.dot_general(mean, wl_ref[...], (((1,), (1,)), ((), ())), **_DOT)
  h = h + lax.dot_general(xs, wr_ref[...], (((1,), (1,)), ((), ())), **_DOT)
  h = jnp.maximum(h + bl_ref[...], 0.0) + xs
  mu = jnp.mean(h, axis=1, keepdims=True)
  var = jnp.mean((h - mu) ** 2, axis=1, keepdims=True)
  y = (h - mu) * lax.rsqrt(var + 1e-5) * g_ref[...] + be_ref[...]
  h_ref[...] = y

  @pl.when(i == 0)
  def _():
    gsum[...] = jnp.zeros_like(gsum)
    gcnt[...] = jnp.zeros_like(gcnt)

  oh = (b_ref[...] == lax.broadcasted_iota(jnp.int32, (_BR, G), 1))
  oh = oh.astype(jnp.float32)
  gsum[...] += lax.dot_general(oh, y, (((0,), (0,)), ((), ())), **_DOT)
  gcnt[...] += lax.dot_general(oh, jnp.ones((_BR, G), jnp.float32),
                               (((0,), (0,)), ((), ())), **_DOT)

  @pl.when(i == _GRID - 1)
  def _():
    avg = gsum[...] / jnp.maximum(gcnt[...], 1.0)
    avg_ref[...] = avg
    out_ref[...] = lax.dot_general(avg, wc_ref[...], (((1,), (1,)), ((), ())),
                                   **_DOT) + bc_ref[...]


_tc_layer3_pool = pl.pallas_call(
    _layer3_pool_body,
    grid=(_GRID,),
    in_specs=[
        pl.BlockSpec((2, _BR, D), lambda i: (0, i, 0)),
        pl.BlockSpec((_BR, D), lambda i: (i, 0)),
        pl.BlockSpec((_BR, D), lambda i: (i, 0)),
        pl.BlockSpec((D, D), lambda i: (0, 0)),
        pl.BlockSpec((1, D), lambda i: (0, 0)),
        pl.BlockSpec((D, D), lambda i: (0, 0)),
        pl.BlockSpec((1, D), lambda i: (0, 0)),
        pl.BlockSpec((1, D), lambda i: (0, 0)),
        pl.BlockSpec((_BR, 1), lambda i: (i, 0)),
        pl.BlockSpec((C, D), lambda i: (0, 0)),
        pl.BlockSpec((1, C), lambda i: (0, 0)),
    ],
    out_specs=[
        pl.BlockSpec((_BR, D), lambda i: (i, 0)),
        pl.BlockSpec((G, C), lambda i: (0, 0)),
        pl.BlockSpec((G, D), lambda i: (0, 0)),
    ],
    out_shape=[
        jax.ShapeDtypeStruct((N, D), jnp.float32),
        jax.ShapeDtypeStruct((G, C), jnp.float32),
        jax.ShapeDtypeStruct((G, D), jnp.float32),
    ],
    scratch_shapes=[
        pltpu.VMEM((G, D), jnp.float32),
        pltpu.VMEM((G, G), jnp.float32),
    ],
)


def kernel(x, edge_index, batch, W1l, b1l, W1r, g1, be1, W2l, b2l, W2r, g2,
           be2, W3l, b3l, W3r, g3, be3, Wc, bc):
  src2 = edge_index[0].reshape(E // EC, EC)
  dst2 = edge_index[1].reshape(E // EC, EC)
  zeros_d = jnp.zeros((624, D), jnp.float32)
  zeros16 = jnp.zeros((624, 16), jnp.float32)
  ones_h = jnp.ones((EC, 16), jnp.float32)

  agg1, cnt1 = _sc_aggregate_cnt(x, src2, dst2, zeros_d, zeros16, ones_h)
  h1, invc = _tc_layer1(agg1.reshape(2, N, D), cnt1.reshape(2, N, D), x,
                        W1l, b1l.reshape(1, D), W1r,
                        g1.reshape(1, D), be1.reshape(1, D))

  agg2 = _sc_aggregate(h1, src2, dst2, zeros_d)[0].reshape(2, N, D)
  h2 = _tc_layer2(agg2, h1, invc, W2l, b2l.reshape(1, D), W2r,
                  g2.reshape(1, D), be2.reshape(1, D))

  agg3 = _sc_aggregate(h2, src2, dst2, zeros_d)[0].reshape(2, N, D)
  h3, out, avg = _tc_layer3_pool(agg3, h2, invc, W3l, b3l.reshape(1, D),
                                 W3r, g3.reshape(1, D), be3.reshape(1, D),
                                 batch.reshape(N, 1), Wc, bc.reshape(1, C))
  return (out, h3, avg)
